# baseline (speedup 1.0000x reference)
import functools

import jax
import jax.numpy as jnp
from jax import lax
from jax.experimental import pallas as pl
from jax.experimental.pallas import tpu as pltpu
from jax.experimental.pallas import tpu_sc as plsc

_NC = 2
_NS = 16
_NW = _NC * _NS
_L = 16

_N_OUT = 128 * 512 * 3 * 33
_N_IN = 2 * _N_OUT
_PER_W = _N_OUT // _NW
_NBLK = 12
_BO = _PER_W // _NBLK
_BI = 2 * _BO


def _body(x_hbm, out_hbm, in_v, out_v):
    c = lax.axis_index("c")
    s = lax.axis_index("s")
    wid = s * _NC + c
    base = wid * _PER_W
    even_idx = lax.iota(jnp.int32, _L) * 2

    def block(j, carry):
        ob = base + j * _BO
        pltpu.sync_copy(x_hbm.at[pl.ds(2 * ob, _BI)], in_v)

        def inner(v, carry2):
            idx = even_idx + v * (2 * _L)
            e = plsc.load_gather(in_v, [idx])
            o = plsc.load_gather(in_v, [idx + 1])
            out_v[pl.ds(v * _L, _L)] = e + o
            return carry2

        lax.fori_loop(0, _BO // _L, inner, 0)
        pltpu.sync_copy(out_v, out_hbm.at[pl.ds(ob, _BO)])
        return carry

    lax.fori_loop(0, _NBLK, block, 0)


_pool = pl.kernel(
    _body,
    out_type=jax.ShapeDtypeStruct((_N_OUT,), jnp.float32),
    mesh=plsc.VectorSubcoreMesh(
        core_axis_name="c", subcore_axis_name="s",
        num_cores=_NC, num_subcores=_NS,
    ),
    scratch_types=[
        pltpu.VMEM((_BI,), jnp.float32),
        pltpu.VMEM((_BO,), jnp.float32),
    ],
    compiler_params=pltpu.CompilerParams(needs_layout_passes=False),
)


@jax.jit
def kernel(x):
    flat = x.reshape(_N_IN)
    return _pool(flat).reshape(128, 512, 3, 33)


# speedup vs baseline: 1.2201x; 1.2201x over previous
import jax
import jax.numpy as jnp
from jax import lax
from jax.experimental import pallas as pl
from jax.experimental.pallas import tpu as pltpu
from jax.experimental.pallas import tpu_sc as plsc

_NC = 2
_NS = 16
_NW = _NC * _NS
_L = 16

_N_OUT = 128 * 512 * 3 * 33
_N_IN = 2 * _N_OUT
_PER_W = _N_OUT // _NW
_NBLK = 12
_BO = _PER_W // _NBLK
_BI = 2 * _BO


def _body(x_hbm, out_hbm, in0, in1, o0, o1, si0, si1, so0, so1):
    c = lax.axis_index("c")
    s = lax.axis_index("s")
    wid = s * _NC + c
    base = wid * _PER_W
    even_idx = lax.iota(jnp.int32, _L) * 2

    ins = (in0, in1)
    outs = (o0, o1)
    sis = (si0, si1)
    sos = (so0, so1)

    def start_in(j):
        ob = base + j * _BO
        return pltpu.async_copy(
            x_hbm.at[pl.ds(2 * ob, _BI)], ins[j % 2], sis[j % 2])

    pending_in = [start_in(0), None]
    pending_out = [None, None]
    for j in range(_NBLK):
        if j + 1 < _NBLK:
            pending_in[(j + 1) % 2] = start_in(j + 1)
        pending_in[j % 2].wait()
        if pending_out[j % 2] is not None:
            pending_out[j % 2].wait()
        in_v = ins[j % 2]
        out_v = outs[j % 2]

        @plsc.parallel_loop(0, _BO, _L, unroll=8)
        def block(v):
            idx = even_idx + 2 * v
            e = plsc.load_gather(in_v, [idx])
            o = plsc.load_gather(in_v, [idx + 1])
            out_v[pl.ds(v, _L)] = e + o

        ob = base + j * _BO
        pending_out[j % 2] = pltpu.async_copy(
            out_v, out_hbm.at[pl.ds(ob, _BO)], sos[j % 2])

    pending_out[0].wait()
    pending_out[1].wait()


_pool = pl.kernel(
    _body,
    out_type=jax.ShapeDtypeStruct((_N_OUT,), jnp.float32),
    mesh=plsc.VectorSubcoreMesh(
        core_axis_name="c", subcore_axis_name="s",
        num_cores=_NC, num_subcores=_NS,
    ),
    scratch_types=[
        pltpu.VMEM((_BI,), jnp.float32),
        pltpu.VMEM((_BI,), jnp.float32),
        pltpu.VMEM((_BO,), jnp.float32),
        pltpu.VMEM((_BO,), jnp.float32),
        pltpu.SemaphoreType.DMA,
        pltpu.SemaphoreType.DMA,
        pltpu.SemaphoreType.DMA,
        pltpu.SemaphoreType.DMA,
    ],
    compiler_params=pltpu.CompilerParams(needs_layout_passes=False),
)


@jax.jit
def kernel(x):
    flat = x.reshape(_N_IN)
    return _pool(flat).reshape(128, 512, 3, 33)
